# baseline (speedup 1.0000x reference)
import functools

import jax
import jax.numpy as jnp
from jax import lax
from jax.experimental import pallas as pl
from jax.experimental.pallas import tpu as pltpu
from jax.experimental.pallas import tpu_sc as plsc

_F = 26
_V = 1_000_000
_B = 16384
_NC, _NS, _L = 2, 16, 16
_NW = _NC * _NS
_BPW = _B // _NW
_IPW = _BPW * _F

_WIN = 1024
_NFULL = _V // _WIN
_REM = _V - _NFULL * _WIN
_REM_A = (_REM // 128) * 128
_REM_B = _REM - _REM_A

_mesh = plsc.VectorSubcoreMesh(
    core_axis_name="c", subcore_axis_name="s", num_cores=_NC, num_subcores=_NS
)


@functools.partial(
    pl.kernel,
    out_type=jax.ShapeDtypeStruct((_F * _V,), jnp.float32),
    mesh=_mesh,
    scratch_types=[
        pltpu.VMEM((_F, _WIN), jnp.float32),
        pltpu.VMEM((_F, _WIN), jnp.float32),
        pltpu.VMEM((_F * _WIN,), jnp.float32),
        pltpu.VMEM((_F * _WIN,), jnp.float32),
        pltpu.SemaphoreType.DMA,
        pltpu.SemaphoreType.DMA,
        pltpu.SemaphoreType.DMA,
        pltpu.SemaphoreType.DMA,
    ],
)
def _detile_kernel(table_hbm, tail_hbm, lin_hbm, stripe_a, stripe_b, rows_a,
                   rows_b, isem_a, isem_b, osem_a, osem_b):
    wid = lax.axis_index("s") * _NC + lax.axis_index("c")
    _ROUNDS = (_NFULL + _NW - 1) // _NW

    def _fetch(w, buf, sem, width=_WIN):
        off = pl.multiple_of(w * _WIN, _WIN)
        return pltpu.async_copy(
            table_hbm.at[:, pl.ds(off, width)],
            buf.at[:, pl.ds(0, width)],
            sem,
        )

    def _wait_fetch(w, buf, sem, width=_WIN):
        off = pl.multiple_of(w * _WIN, _WIN)
        pltpu.make_async_copy(
            table_hbm.at[:, pl.ds(off, width)],
            buf.at[:, pl.ds(0, width)],
            sem,
        ).wait()

    def _drain_outs(rows_v, sem):
        pltpu.make_async_copy(
            lin_hbm.at[pl.ds(0, _F * _WIN)], rows_v, sem
        ).wait()

    def _extract(w, buf, rows_v, sem, width=_WIN):
        off = pl.multiple_of(w * _WIN, _WIN)
        for r in range(_F):

            @pl.loop(0, width // _L, unroll=16)
            def _(c):
                rows_v[pl.ds(r * _WIN + c * _L, _L)] = buf[r, pl.ds(c * _L, _L)]

            pltpu.async_copy(
                rows_v.at[pl.ds(r * _WIN, width)],
                lin_hbm.at[pl.ds(r * _V + off, width)],
                sem,
            )

    def _round(k, cur, nxt, my_isem, nxt_isem, rows_v, my_osem):
        w = k * _NW + wid

        @pl.when(w < _NFULL)
        def _():
            w2 = w + _NW

            @pl.when(w2 < _NFULL)
            def _():
                _fetch(w2, nxt, nxt_isem)

            _wait_fetch(w, cur, my_isem)

            @pl.when(w >= 2 * _NW)
            def _():
                _drain_outs(rows_v, my_osem)

            _extract(w, cur, rows_v, my_osem)

    _fetch(wid, stripe_a, isem_a)

    @pl.loop(0, _ROUNDS + (_ROUNDS % 2), step=2)
    def _(k):
        _round(k, stripe_a, stripe_b, isem_a, isem_b, rows_a, osem_a)
        _round(k + 1, stripe_b, stripe_a, isem_b, isem_a, rows_b, osem_b)

    _drain_outs(rows_a, osem_a)
    _drain_outs(rows_b, osem_b)

    @pl.when(wid == 0)
    def _():
        _fetch(_NFULL, stripe_a, isem_a, _REM_A).wait()
        _extract(_NFULL, stripe_a, rows_a, osem_a, _REM_A)
        pltpu.make_async_copy(
            lin_hbm.at[pl.ds(0, _F * _REM_A)],
            rows_a.at[pl.ds(0, _F * _REM_A)],
            osem_a,
        ).wait()

    @pl.when(wid == 1)
    def _():
        pltpu.sync_copy(tail_hbm, rows_a.at[pl.ds(0, _F * _REM_B)])
        for r in range(_F):
            pltpu.async_copy(
                rows_a.at[pl.ds(r * _REM_B, _REM_B)],
                lin_hbm.at[pl.ds(r * _V + _NFULL * _WIN + _REM_A, _REM_B)],
                osem_a,
            ).wait()


@functools.partial(
    pl.kernel,
    out_type=jax.ShapeDtypeStruct((_B,), jnp.float32),
    mesh=_mesh,
    scratch_types=[
        pltpu.VMEM((_IPW,), jnp.int32),
        pltpu.VMEM((_IPW,), jnp.float32),
        pltpu.VMEM((_BPW,), jnp.float32),
        pltpu.SemaphoreType.DMA,
    ],
)
def _gather_kernel(idx_hbm, lin_hbm, out_hbm, idx_v, vals_v, out_v, sem):
    wid = lax.axis_index("s") * _NC + lax.axis_index("c")
    pltpu.sync_copy(idx_hbm.at[wid], idx_v)
    pltpu.async_copy(lin_hbm.at[idx_v], vals_v, sem).wait()

    for j in range(_BPW // _L):
        acc = None
        for f in range(_F):
            v = vals_v[pl.ds(f * _BPW + j * _L, _L)]
            acc = v if acc is None else acc + v
        out_v[pl.ds(j * _L, _L)] = 1.0 / (1.0 + jnp.exp(-acc))

    pltpu.sync_copy(out_v, out_hbm.at[pl.ds(wid * _BPW, _BPW)])


def kernel(X, lin_table):
    offs = jnp.arange(_F, dtype=jnp.int32) * _V
    idx = (X + offs[None, :]).T.reshape(_F, _NW, _BPW).transpose(1, 0, 2)
    idx = idx.reshape(_NW, _IPW)
    tail = lin_table[:, _V - _REM_B:].reshape(-1)
    lin = _detile_kernel(lin_table, tail)
    out = _gather_kernel(idx, lin)
    return out.reshape(_B, 1)
